# baseline (speedup 1.0000x reference)
import jax
import jax.numpy as jnp
import numpy as np
from jax import lax
from jax.experimental import pallas as pl
from jax.experimental.pallas import tpu as pltpu

H = 64
W = 64
A = 9
N = H * W * A
B = 4
K = 50
RR = 288
LL = 128
NEG_OV = 0.3
POS_OV = 0.7
NUM_FG = 128.0
MAX_LABELS = 256.0


def _base_anchors():
    base_size = 16.0
    ratios = np.array([0.5, 1.0, 2.0])
    scales = np.array([8.0, 16.0, 32.0])
    w = h = base_size
    cx = cy = 0.5 * (base_size - 1.0)
    size = w * h
    ws = np.round(np.sqrt(size / ratios))
    hs = np.round(ws * ratios)
    anchors = []
    for i in range(len(ratios)):
        for s in scales:
            W_ = ws[i] * s
            H_ = hs[i] * s
            anchors.append([cx - 0.5 * (W_ - 1), cy - 0.5 * (H_ - 1),
                            cx + 0.5 * (W_ - 1), cy + 0.5 * (H_ - 1)])
    return np.array(anchors, dtype=np.float32)


def _all_anchors():
    base = _base_anchors()
    sx = np.arange(W) * 16
    sy = np.arange(H) * 16
    sxx, syy = np.meshgrid(sx, sy)
    shifts = np.stack([sxx.ravel(), syy.ravel(), sxx.ravel(), syy.ravel()],
                      axis=1).astype(np.float32)
    return (shifts[:, None, :] + base[None, :, :]).reshape(-1, 4)


def _to_out_major(x):
    lead = x.shape[:-1]
    x = x.reshape(lead + (H, W, A))
    x = np.moveaxis(x, -1, -3)
    return np.ascontiguousarray(x).reshape(lead + (RR, LL))


_ANC = _to_out_major(np.ascontiguousarray(_all_anchors().T))

def _rotl32(x, r):
    return ((x << np.uint32(r)) | (x >> np.uint32(32 - r))).astype(np.uint32)


def _threefry2x32(k0, k1, x0, x1):
    x0 = x0.astype(np.uint32).copy()
    x1 = x1.astype(np.uint32).copy()
    rotations = ((13, 15, 26, 6), (17, 29, 16, 24))
    ks = (np.uint32(k0), np.uint32(k1),
          np.uint32(np.uint32(0x1BD11BDA) ^ np.uint32(k0) ^ np.uint32(k1)))
    x0 = (x0 + ks[0]).astype(np.uint32)
    x1 = (x1 + ks[1]).astype(np.uint32)
    for i in range(5):
        for r in rotations[i % 2]:
            x0 = (x0 + x1).astype(np.uint32)
            x1 = (x0 ^ _rotl32(x1, r)).astype(np.uint32)
        x0 = (x0 + ks[(i + 1) % 3]).astype(np.uint32)
        x1 = (x1 + ks[(i + 2) % 3] + np.uint32(i + 1)).astype(np.uint32)
    return x0, x1


def _fixed_uniform_bits():
    b1, b2 = _threefry2x32(np.uint32(0), np.uint32(42),
                           np.zeros(2, np.uint32), np.arange(2, dtype=np.uint32))
    keys = np.stack([b1, b2], axis=1)
    out = []
    for k0, k1 in keys:
        hi = np.zeros(B * N, np.uint32)
        lo = np.arange(B * N, dtype=np.uint32)
        r0, r1 = _threefry2x32(k0, k1, hi, lo)
        bits = (r0 ^ r1).astype(np.uint32)
        m = (bits >> np.uint32(9)).astype(np.int32)
        out.append(_to_out_major(m.reshape(B, N)))
    return out


_BFG, _BBG = _fixed_uniform_bits()


def _body(scal_ref, gts_ref, anc_ref, bfg_ref, bbg_ref,
          lab_ref, bt_ref, biw_ref, bow_ref,
          insf_s, aa_s, mov_s, kc_s, scx_s, scy_s, sw_s, sh_s, cnt_s):
    im_h = scal_ref[0]
    im_w = scal_ref[1]
    one = scal_ref[2]

    row_i = lax.broadcasted_iota(jnp.int32, (RR, LL), 0)
    lane_i = lax.broadcasted_iota(jnp.int32, (RR, LL), 1)
    q_i = row_i * LL + lane_i
    idx_arr = (((q_i // W) % H) * W + q_i % W) * A + q_i // (H * W)

    ax1 = anc_ref[0]
    ay1 = anc_ref[1]
    ax2 = anc_ref[2]
    ay2 = anc_ref[3]
    insf_s[...] = jnp.where((ax1 >= 0.0) & (ay1 >= 0.0)
                            & (ax2 < im_w) & (ay2 < im_h), 1.0, 0.0)
    aw0 = ax2 - ax1 + 1.0
    ah0 = ay2 - ay1 + 1.0
    aa_s[...] = aw0 * ah0

    def batch_body(b, _):
        mov_s[...] = jnp.full((RR, LL), -3.0, dtype=jnp.float32)
        kc_s[...] = jnp.zeros((RR, LL), dtype=jnp.float32)

        def k_body(k2, __):
            gvals = []
            for dk in range(10):
                k = k2 * 10 + dk
                gx1 = gts_ref[b, k, 0]
                gy1 = gts_ref[b, k, 1]
                gx2 = gts_ref[b, k, 2]
                gy2 = gts_ref[b, k, 3]
                gw = gx2 - gx1 + 1.0
                gh = gy2 - gy1 + 1.0
                g_area = gw * gh
                gcx = gx1 + 0.5 * gw
                gcy = gy1 + 0.5 * gh
                iw = jnp.maximum(jnp.minimum(anc_ref[2], gx2)
                                 - jnp.maximum(anc_ref[0], gx1) + 1.0, 0.0)
                ih = jnp.maximum(jnp.minimum(anc_ref[3], gy2)
                                 - jnp.maximum(anc_ref[1], gy1) + 1.0, 0.0)
                inter = iw * ih
                union = aa_s[...] + g_area - inter
                iou = inter / union
                masked = jnp.where(insf_s[...] > 0.0, iou, -1.0)
                m = jnp.max(masked, axis=(0, 1), keepdims=True)
                gadj = jnp.where(m == 0.0, 1e-5, m)
                gvals.append((masked, gadj, gcx, gcy, gw, gh))

            kc = kc_s[...]
            for masked, gadj, *_ in gvals:
                kc = kc + jnp.where(masked == gadj, 1.0, 0.0)
            kc_s[...] = kc
            mp = mov_s[...]
            scx = scx_s[...]
            scy = scy_s[...]
            sw = sw_s[...]
            sh = sh_s[...]
            for masked, _, gcx, gcy, gw, gh in gvals:
                upd = masked > mp
                mp = jnp.where(upd, masked, mp)
                scx = jnp.where(upd, gcx, scx)
                scy = jnp.where(upd, gcy, scy)
                sw = jnp.where(upd, gw, sw)
                sh = jnp.where(upd, gh, sh)
            mov_s[...] = mp
            scx_s[...] = scx
            scy_s[...] = scy
            sw_s[...] = sw
            sh_s[...] = sh
            return 0

        lax.fori_loop(0, K // 10, k_body, 0)

        ins = insf_s[...] > 0.0
        mov = mov_s[...]
        keep = kc_s[...]
        lab = jnp.full((RR, LL), -1.0, dtype=jnp.float32)
        lab = jnp.where(ins & (mov < NEG_OV), 0.0, lab)
        lab = jnp.where(ins & (keep > 0.0), 1.0, lab)
        lab = jnp.where(ins & (mov >= POS_OV), 1.0, lab)
        cnt_s[b] = jnp.sum(jnp.where(lab == 1.0, 1.0, 0.0))
        cnt_s[b + B] = jnp.sum(jnp.where(lab == 0.0, 1.0, 0.0))
        lab_ref[b] = lab

        ax1 = anc_ref[0]
        ay1 = anc_ref[1]
        aw = anc_ref[2] - ax1 + 1.0
        ah = anc_ref[3] - ay1 + 1.0
        acx = ax1 + 0.5 * aw
        acy = ay1 + 0.5 * ah
        dx = jnp.where(ins, (scx_s[...] - acx) / aw, 0.0) * one
        dy = jnp.where(ins, (scy_s[...] - acy) / ah, 0.0) * one
        dw = jnp.where(ins, jnp.log(sw_s[...] / aw), 0.0) * one
        dh = jnp.where(ins, jnp.log(sh_s[...] / ah), 0.0) * one
        for a in range(A):
            sl = slice(a * 32, (a + 1) * 32)
            bt_ref[b, a * 4 + 0] = dx[sl]
            bt_ref[b, a * 4 + 1] = dy[sl]
            bt_ref[b, a * 4 + 2] = dw[sl]
            bt_ref[b, a * 4 + 3] = dh[sl]
        return 0

    lax.fori_loop(0, B, batch_body, 0)

    cfgs = [cnt_s[b] for b in range(B)]
    cbgs = [cnt_s[b + B] for b in range(B)]
    tbgs = [MAX_LABELS - jnp.minimum(cfgs[b], NUM_FG) for b in range(B)]
    targets = [jnp.float32(NUM_FG)] * B + tbgs
    clsvals = [1.0] * B + [0.0] * B
    bit_refs = [bfg_ref] * B + [bbg_ref] * B

    def masks_bits(i):
        b = i % B
        return (lab_ref[b] == clsvals[i]), bit_refs[i][b]

    mkeys = []
    for i in range(2 * B):
        clsm, bits = masks_bits(i)
        mkeys.append(jnp.where(clsm, bits, -1))

    def cnt_ge(i, x):
        return jnp.sum(jnp.where(mkeys[i] >= x, 1.0, 0.0))

    def vstep(_, lhs):
        out = []
        for i in range(2 * B):
            lo, hi = lhs[i]
            mid = lo + (hi - lo + 1) // 2
            ok = cnt_ge(i, mid) >= targets[i]
            out.append((jnp.where(ok, mid, lo), jnp.where(ok, hi, mid - 1)))
        return tuple(out)

    init = tuple((jnp.int32(0), jnp.int32(1 << 23)) for _ in range(2 * B))
    lhs = lax.fori_loop(0, 24, vstep, init)
    ts = [lhs[i][0] for i in range(2 * B)]
    tie_targets = [targets[i] - cnt_ge(i, ts[i] + 1) for i in range(2 * B)]
    ikeys = [jnp.where(mkeys[i] == ts[i], idx_arr, N) for i in range(2 * B)]

    def istep(_, lhs):
        out = []
        for i in range(2 * B):
            lo, hi = lhs[i]
            mid = (lo + hi) // 2
            c = jnp.sum(jnp.where(ikeys[i] <= mid, 1.0, 0.0))
            ok = c >= tie_targets[i]
            out.append((jnp.where(ok, lo, mid + 1), jnp.where(ok, mid, hi)))
        return tuple(out)

    init2 = tuple((jnp.int32(0), jnp.int32(N - 1)) for _ in range(2 * B))
    lhs2 = lax.fori_loop(0, 16, istep, init2)
    idx_ts = [lhs2[i][0] for i in range(2 * B)]

    kept_fg3 = jnp.minimum(cfgs[B - 1], NUM_FG)
    kept_bg3 = jnp.minimum(cbgs[B - 1], tbgs[B - 1])
    pw = 1.0 / (kept_fg3 + kept_bg3)

    for b in range(B):
        labarr = lab_ref[b]
        for i in (b, b + B):
            keep = (mkeys[i] > ts[i]) | ((mkeys[i] == ts[i])
                                         & (idx_arr <= idx_ts[i]))
            clsm = lab_ref[b] == clsvals[i]
            labarr = jnp.where(clsm & jnp.logical_not(keep), -1.0, labarr)
        lab_ref[b] = labarr * one
        biw = jnp.where(labarr == 1.0, 1.0, 0.0) * one
        bow = jnp.where(labarr >= 0.0, pw, 0.0) * one
        for a in range(A):
            bblk = biw[a * 32:(a + 1) * 32]
            oblk = bow[a * 32:(a + 1) * 32]
            for j in range(4):
                biw_ref[b, a * 4 + j] = bblk
                bow_ref[b, a * 4 + j] = oblk


def kernel(input0, gt_boxes, im_info):
    gts = gt_boxes[:, :, :4].astype(jnp.float32)
    hw = input0[2] + input0[3]
    one = (hw // hw).astype(jnp.float32)
    scal = jnp.stack([im_info[0, 0], im_info[0, 1], one,
                      jnp.float32(0.0)]).astype(jnp.float32)

    lab, bt, biw, bow = pl.pallas_call(
        _body,
        out_shape=[
            jax.ShapeDtypeStruct((B, RR, LL), jnp.float32),
            jax.ShapeDtypeStruct((B, 4 * A, 32, 128), jnp.float32),
            jax.ShapeDtypeStruct((B, 4 * A, 32, 128), jnp.float32),
            jax.ShapeDtypeStruct((B, 4 * A, 32, 128), jnp.float32),
        ],
        in_specs=[
            pl.BlockSpec(memory_space=pltpu.SMEM),
            pl.BlockSpec(memory_space=pltpu.SMEM),
            pl.BlockSpec(memory_space=pltpu.VMEM),
            pl.BlockSpec(memory_space=pltpu.VMEM),
            pl.BlockSpec(memory_space=pltpu.VMEM),
        ],
        out_specs=[
            pl.BlockSpec(memory_space=pltpu.VMEM),
            pl.BlockSpec(memory_space=pltpu.VMEM),
            pl.BlockSpec(memory_space=pltpu.VMEM),
            pl.BlockSpec(memory_space=pltpu.VMEM),
        ],
        scratch_shapes=[
            pltpu.VMEM((RR, LL), jnp.float32),
            pltpu.VMEM((RR, LL), jnp.float32),
            pltpu.VMEM((RR, LL), jnp.float32),
            pltpu.VMEM((RR, LL), jnp.float32),
            pltpu.VMEM((RR, LL), jnp.float32),
            pltpu.VMEM((RR, LL), jnp.float32),
            pltpu.VMEM((RR, LL), jnp.float32),
            pltpu.VMEM((RR, LL), jnp.float32),
            pltpu.SMEM((2 * B,), jnp.float32),
        ],
    )(scal, gts, jnp.asarray(_ANC), jnp.asarray(_BFG), jnp.asarray(_BBG))

    labels_out = lab.reshape(B, 1, A * H, W)
    return (labels_out, bt.reshape(B, 4 * A, H, W),
            biw.reshape(B, 4 * A, H, W), bow.reshape(B, 4 * A, H, W))
